# baseline (speedup 1.0000x reference)
import jax
import jax.numpy as jnp
from jax.experimental import pallas as pl
from jax.experimental.pallas import tpu as pltpu

_B, _D = 65536, 512
_BLK = 1024
_NJ = _B // _BLK


def _body(fs_ref, ft_ref, out_ref, acc_ref):
    j = pl.program_id(0)

    @pl.when(j == 0)
    def _init():
        acc_ref[...] = jnp.zeros_like(acc_ref)

    prod = fs_ref[...] * ft_ref[...]
    acc_ref[...] += jnp.sum(prod.reshape(_BLK // 8, 8, _D), axis=0)

    @pl.when(j == _NJ - 1)
    def _fini():
        cols = jnp.sum(acc_ref[...], axis=0, keepdims=True)
        out_ref[...] = -jnp.sum(cols, axis=1, keepdims=True) / _B


def kernel(f_s, f_t):
    in_spec = pl.BlockSpec((_BLK, _D), lambda j: (j, 0))
    loss = pl.pallas_call(
        _body,
        grid=(_NJ,),
        in_specs=[in_spec, in_spec],
        out_specs=pl.BlockSpec((1, 1), lambda j: (0, 0)),
        out_shape=jax.ShapeDtypeStruct((1, 1), jnp.float32),
        scratch_shapes=[pltpu.VMEM((8, _D), jnp.float32)],
        compiler_params=pltpu.CompilerParams(
            dimension_semantics=("arbitrary",),
        ),
    )(f_s, f_t)
    return loss.reshape(1)


# speedup vs baseline: 1.0886x; 1.0886x over previous
import jax
import jax.numpy as jnp
from jax.experimental import pallas as pl
from jax.experimental.pallas import tpu as pltpu

_B, _D = 65536, 512
_BLK = 2048
_NJ = _B // _BLK
_DEPTH = 4


def _body(fs_hbm, ft_hbm, out_ref, fs_buf, ft_buf, acc_ref, fs_sem, ft_sem):
    def _start(c, slot):
        pltpu.make_async_copy(
            fs_hbm.at[pl.ds(c * _BLK, _BLK), :], fs_buf.at[slot], fs_sem.at[slot]
        ).start()
        pltpu.make_async_copy(
            ft_hbm.at[pl.ds(c * _BLK, _BLK), :], ft_buf.at[slot], ft_sem.at[slot]
        ).start()

    for d in range(_DEPTH):
        _start(d, d)

    acc_ref[...] = jnp.zeros_like(acc_ref)

    def _step(j, _):
        slot = jax.lax.rem(j, _DEPTH)
        pltpu.make_async_copy(
            fs_buf.at[slot], fs_buf.at[slot], fs_sem.at[slot]
        ).wait()
        pltpu.make_async_copy(
            ft_buf.at[slot], ft_buf.at[slot], ft_sem.at[slot]
        ).wait()
        prod = fs_buf[slot] * ft_buf[slot]
        acc_ref[...] += jnp.sum(prod.reshape(_BLK // 8, 8, _D), axis=0)

        nxt = j + _DEPTH

        @pl.when(nxt < _NJ)
        def _():
            _start(nxt, slot)

        return 0

    jax.lax.fori_loop(0, _NJ, _step, 0)

    cols = jnp.sum(acc_ref[...], axis=0, keepdims=True)
    out_ref[...] = -jnp.sum(cols, axis=1, keepdims=True) / _B


def kernel(f_s, f_t):
    loss = pl.pallas_call(
        _body,
        in_specs=[
            pl.BlockSpec(memory_space=pl.ANY),
            pl.BlockSpec(memory_space=pl.ANY),
        ],
        out_specs=pl.BlockSpec(memory_space=pltpu.VMEM),
        out_shape=jax.ShapeDtypeStruct((1, 1), jnp.float32),
        scratch_shapes=[
            pltpu.VMEM((_DEPTH, _BLK, _D), jnp.float32),
            pltpu.VMEM((_DEPTH, _BLK, _D), jnp.float32),
            pltpu.VMEM((8, _D), jnp.float32),
            pltpu.SemaphoreType.DMA((_DEPTH,)),
            pltpu.SemaphoreType.DMA((_DEPTH,)),
        ],
    )(f_s, f_t)
    return loss.reshape(1)


# speedup vs baseline: 1.0898x; 1.0011x over previous
import jax
import jax.numpy as jnp
from jax.experimental import pallas as pl
from jax.experimental.pallas import tpu as pltpu

_B, _D = 65536, 512
_BLK = 2048
_NJ = _B // _BLK


def _body(fs_ref, ft_ref, out_ref, acc_ref):
    j = pl.program_id(0)

    @pl.when(j == 0)
    def _init():
        acc_ref[...] = jnp.zeros_like(acc_ref)

    prod = fs_ref[...] * ft_ref[...]
    acc_ref[...] += jnp.sum(prod.reshape(_BLK // 8, 8, _D), axis=0)

    @pl.when(j == _NJ - 1)
    def _fini():
        cols = jnp.sum(acc_ref[...], axis=0, keepdims=True)
        out_ref[...] = -jnp.sum(cols, axis=1, keepdims=True) / _B


def kernel(f_s, f_t):
    in_spec = pl.BlockSpec((_BLK, _D), lambda j: (j, 0))
    loss = pl.pallas_call(
        _body,
        grid=(_NJ,),
        in_specs=[in_spec, in_spec],
        out_specs=pl.BlockSpec((1, 1), lambda j: (0, 0)),
        out_shape=jax.ShapeDtypeStruct((1, 1), jnp.float32),
        scratch_shapes=[pltpu.VMEM((8, _D), jnp.float32)],
        compiler_params=pltpu.CompilerParams(
            dimension_semantics=("arbitrary",),
        ),
    )(f_s, f_t)
    return loss.reshape(1)
